# baseline (speedup 1.0000x reference)
import jax
import jax.numpy as jnp
from jax import lax
from jax.experimental import pallas as pl
from jax.experimental.pallas import tpu as pltpu
from jax.experimental.pallas import tpu_sc as plsc

N = 10000
E = 320000
D = 128
H = 128

NC = 2
NS = 16
NW = NC * NS
C = 128
NCH = E // C
FULL = NCH // NW
EXTRA = NCH - FULL * NW
NP = 10240
RCH = 128
NRCH = NP // RCH
CPT = NRCH // NS

_f32 = jnp.float32


def _zero_acc(sid, zsrc_v, acc_sh, sem):
    for k in range(CPT):
        j = sid * CPT + k
        pltpu.async_copy(zsrc_v, acc_sh.at[pl.ds(j * RCH, RCH)], sem)
    for k in range(CPT):
        j = sid * CPT + k
        pltpu.make_async_copy(zsrc_v, acc_sh.at[pl.ds(j * RCH, RCH)],
                              sem).wait()


def _copy_out(cid, sid, acc_sh, acc_hbm):
    def obody(k, carry):
        j = sid * CPT + k
        pltpu.sync_copy(acc_sh.at[pl.ds(j * RCH, RCH)],
                        acc_hbm.at[pl.ds(cid * NP + j * RCH, RCH)])
        return carry

    lax.fori_loop(0, CPT, obody, 0)


def _sc_agg_body(y_hbm, ei_hbm, zrow_hbm, acc_hbm,
                 idx_a, rows_a, idx_b, rows_b,
                 acc_sh, sem_a, sem_b, ssem_a, ssem_b):
    cid = lax.axis_index("c")
    sid = lax.axis_index("s")
    wid = cid * NS + sid

    pltpu.sync_copy(zrow_hbm, rows_a)
    _zero_acc(sid, rows_a, acc_sh, sem_a)
    plsc.subcore_barrier()

    def load_and_gather(i, idx_v, rows_v, gsem):
        b = (wid + i * NW) * C
        pltpu.sync_copy(ei_hbm.at[:, pl.ds(b, C)], idx_v)
        pltpu.async_copy(y_hbm.at[idx_v.at[0]], rows_v, gsem)

    def gwait_and_scatter(idx_v, rows_v, gsem, ssem):
        pltpu.make_async_copy(y_hbm.at[idx_v.at[0]], rows_v, gsem).wait()
        pltpu.async_copy(rows_v, acc_sh.at[idx_v.at[1]], ssem, add=True)

    def swait(idx_v, rows_v, ssem):
        pltpu.make_async_copy(rows_v, acc_sh.at[idx_v.at[1]], ssem).wait()

    load_and_gather(0, idx_a, rows_a, sem_a)
    gwait_and_scatter(idx_a, rows_a, sem_a, ssem_a)
    load_and_gather(1, idx_b, rows_b, sem_b)
    gwait_and_scatter(idx_b, rows_b, sem_b, ssem_b)
    swait(idx_a, rows_a, ssem_a)
    load_and_gather(2, idx_a, rows_a, sem_a)

    def ebody(k, carry):
        i = 2 * k
        gwait_and_scatter(idx_a, rows_a, sem_a, ssem_a)
        swait(idx_b, rows_b, ssem_b)
        load_and_gather(i + 1, idx_b, rows_b, sem_b)
        gwait_and_scatter(idx_b, rows_b, sem_b, ssem_b)
        swait(idx_a, rows_a, ssem_a)
        load_and_gather(i + 2, idx_a, rows_a, sem_a)
        return carry

    lax.fori_loop(1, FULL // 2 - 1, ebody, 0)
    gwait_and_scatter(idx_a, rows_a, sem_a, ssem_a)
    swait(idx_b, rows_b, ssem_b)
    load_and_gather(FULL - 1, idx_b, rows_b, sem_b)
    gwait_and_scatter(idx_b, rows_b, sem_b, ssem_b)
    swait(idx_a, rows_a, ssem_a)
    swait(idx_b, rows_b, ssem_b)

    @pl.when(wid < EXTRA)
    def _():
        b = (FULL * NW + wid) * C
        pltpu.sync_copy(ei_hbm.at[:, pl.ds(b, C)], idx_a)
        pltpu.async_copy(y_hbm.at[idx_a.at[0]], rows_a, sem_a)
        pltpu.make_async_copy(y_hbm.at[idx_a.at[0]], rows_a, sem_a).wait()
        pltpu.sync_copy(rows_a, acc_sh.at[idx_a.at[1]], add=True)

    plsc.subcore_barrier()
    _copy_out(cid, sid, acc_sh, acc_hbm)


def _sc_cnt_body(ei_hbm, zrow_hbm, ones_hbm, cnt_hbm,
                 idx_a, idx_b, ones_v, stage_v, acc_sh, sem_a, sem_b):
    cid = lax.axis_index("c")
    sid = lax.axis_index("s")
    wid = cid * NS + sid

    pltpu.sync_copy(zrow_hbm, stage_v)
    pltpu.sync_copy(ones_hbm, ones_v)
    _zero_acc(sid, stage_v, acc_sh, sem_a)
    plsc.subcore_barrier()

    def load(i, idx_v):
        b = (wid + i * NW) * C
        pltpu.sync_copy(ei_hbm.at[:, pl.ds(b, C)], idx_v)

    def scat(idx_v, sem):
        pltpu.async_copy(ones_v, acc_sh.at[idx_v.at[1]], sem, add=True)

    def swait(idx_v, sem):
        pltpu.make_async_copy(ones_v, acc_sh.at[idx_v.at[1]], sem).wait()

    load(0, idx_a)
    scat(idx_a, sem_a)

    def ebody(k, carry):
        i = 2 * k
        load(i + 1, idx_b)
        scat(idx_b, sem_b)
        swait(idx_a, sem_a)
        load(i + 2, idx_a)
        scat(idx_a, sem_a)
        swait(idx_b, sem_b)
        return carry

    lax.fori_loop(0, (FULL - 2) // 2, ebody, 0)
    load(FULL - 1, idx_b)
    scat(idx_b, sem_b)
    swait(idx_a, sem_a)
    swait(idx_b, sem_b)

    @pl.when(wid < EXTRA)
    def _():
        b = (FULL * NW + wid) * C
        pltpu.sync_copy(ei_hbm.at[:, pl.ds(b, C)], idx_a)
        pltpu.sync_copy(ones_v, acc_sh.at[idx_a.at[1]], add=True)

    plsc.subcore_barrier()
    _copy_out(cid, sid, acc_sh, cnt_hbm)


_sc_mesh = plsc.VectorSubcoreMesh(core_axis_name="c", subcore_axis_name="s")

_sc_agg = pl.kernel(
    _sc_agg_body,
    out_type=jax.ShapeDtypeStruct((NC * NP, H), _f32),
    mesh=_sc_mesh,
    scratch_types=[
        pltpu.VMEM((2, C), jnp.int32),
        pltpu.VMEM((C, H), _f32),
        pltpu.VMEM((2, C), jnp.int32),
        pltpu.VMEM((C, H), _f32),
        pltpu.VMEM_SHARED((NP, H), _f32),
        pltpu.SemaphoreType.DMA,
        pltpu.SemaphoreType.DMA,
        pltpu.SemaphoreType.DMA,
        pltpu.SemaphoreType.DMA,
    ],
)

_sc_cnt = pl.kernel(
    _sc_cnt_body,
    out_type=jax.ShapeDtypeStruct((NC * NP, H), _f32),
    mesh=_sc_mesh,
    scratch_types=[
        pltpu.VMEM((2, C), jnp.int32),
        pltpu.VMEM((2, C), jnp.int32),
        pltpu.VMEM((C, H), _f32),
        pltpu.VMEM((RCH, H), _f32),
        pltpu.VMEM_SHARED((NP, H), _f32),
        pltpu.SemaphoreType.DMA,
        pltpu.SemaphoreType.DMA,
    ],
)

_DN = (((1,), (1,)), ((), ()))


def _tc_pre_body(x_ref, wl_ref, wr_ref, b_ref, y_ref, z_ref):
    x = x_ref[...]
    y_ref[...] = lax.dot_general(x, wl_ref[...], _DN,
                                 preferred_element_type=_f32)
    z_ref[...] = lax.dot_general(x, wr_ref[...], _DN,
                                 preferred_element_type=_f32) + b_ref[...]


def _inv_cnt(cnt_ref):
    cnt = cnt_ref[:N, :] + cnt_ref[NP:NP + N, :]
    return 1.0 / jnp.maximum(cnt[:, 0:1], 1.0)


def _tc_mid_body(acc_ref, cnt_ref, z_ref, wl_ref, wr_ref, b_ref,
                 y2_ref, z2_ref):
    acc = acc_ref[:N, :] + acc_ref[NP:NP + N, :]
    h = jnp.maximum(acc * _inv_cnt(cnt_ref) + z_ref[...], 0.0)
    y2_ref[...] = lax.dot_general(h, wl_ref[...], _DN,
                                  preferred_element_type=_f32)
    z2_ref[...] = lax.dot_general(h, wr_ref[...], _DN,
                                  preferred_element_type=_f32) + b_ref[...]


def _tc_post_body(acc_ref, cnt_ref, z_ref, o_ref):
    acc = acc_ref[:N, :] + acc_ref[NP:NP + N, :]
    o_ref[...] = acc * _inv_cnt(cnt_ref) + z_ref[...]


_nh = jax.ShapeDtypeStruct((N, H), _f32)

_tc_pre = pl.pallas_call(_tc_pre_body, out_shape=[_nh, _nh])
_tc_mid = pl.pallas_call(_tc_mid_body, out_shape=[_nh, _nh])
_tc_post = pl.pallas_call(_tc_post_body, out_shape=_nh)


@jax.jit
def kernel(x, edge_index, Wl1, Wr1, b1, Wl2, Wr2, b2):
    zrow = jnp.zeros((RCH, H), _f32)
    ones = jnp.ones((C, H), _f32)

    y1, z1 = _tc_pre(x, Wl1, Wr1, b1.reshape(1, H))
    cntp = _sc_cnt(edge_index, zrow, ones)
    p1 = _sc_agg(y1, edge_index, zrow)
    y2, z2 = _tc_mid(p1, cntp, z1, Wl2, Wr2, b2.reshape(1, H))
    p2 = _sc_agg(y2, edge_index, zrow)
    return _tc_post(p2, cntp, z2)


# speedup vs baseline: 1.5397x; 1.5397x over previous
import jax
import jax.numpy as jnp
from jax import lax
from jax.experimental import pallas as pl
from jax.experimental.pallas import tpu as pltpu
from jax.experimental.pallas import tpu_sc as plsc

N = 10000
E = 320000
D = 128
H = 128

NC = 2
NS = 16
NW = NC * NS
C = 128
NCH = E // C
FULL = NCH // NW
EXTRA = NCH - FULL * NW
NP = 10240
RCH = 128
NRCH = NP // RCH
CPT = NRCH // NS

_f32 = jnp.float32


def _zero_acc(sid, zsrc_v, acc_sh, sem):
    for k in range(CPT):
        j = sid * CPT + k
        pltpu.async_copy(zsrc_v, acc_sh.at[pl.ds(j * RCH, RCH)], sem)
    for k in range(CPT):
        j = sid * CPT + k
        pltpu.make_async_copy(zsrc_v, acc_sh.at[pl.ds(j * RCH, RCH)],
                              sem).wait()


def _copy_out(cid, sid, acc_sh, acc_hbm):
    def obody(k, carry):
        j = sid * CPT + k
        pltpu.sync_copy(acc_sh.at[pl.ds(j * RCH, RCH)],
                        acc_hbm.at[pl.ds(cid * NP + j * RCH, RCH)])
        return carry

    lax.fori_loop(0, CPT, obody, 0)


def _sc_agg_body(y_hbm, ei_hbm, zrow_hbm, acc_hbm,
                 idx_a, rows_a, idx_b, rows_b,
                 acc_sh, sem_a, sem_b):
    cid = lax.axis_index("c")
    sid = lax.axis_index("s")
    wid = cid * NS + sid

    pltpu.sync_copy(zrow_hbm, rows_a)
    _zero_acc(sid, rows_a, acc_sh, sem_a)
    plsc.subcore_barrier()

    def load_and_gather(i, idx_v, rows_v, gsem):
        b = (wid + i * NW) * C
        pltpu.sync_copy(ei_hbm.at[:, pl.ds(b, C)], idx_v)
        pltpu.async_copy(y_hbm.at[idx_v.at[0]], rows_v, gsem)

    def wait_and_scatter(idx_v, rows_v, gsem):
        pltpu.make_async_copy(y_hbm.at[idx_v.at[0]], rows_v, gsem).wait()
        pltpu.sync_copy(rows_v, acc_sh.at[idx_v.at[1]], add=True)

    load_and_gather(0, idx_a, rows_a, sem_a)

    def ebody(k, carry):
        i = 2 * k
        load_and_gather(i + 1, idx_b, rows_b, sem_b)
        wait_and_scatter(idx_a, rows_a, sem_a)
        load_and_gather(i + 2, idx_a, rows_a, sem_a)
        wait_and_scatter(idx_b, rows_b, sem_b)
        return carry

    lax.fori_loop(0, (FULL - 2) // 2, ebody, 0)
    load_and_gather(FULL - 1, idx_b, rows_b, sem_b)
    wait_and_scatter(idx_a, rows_a, sem_a)
    wait_and_scatter(idx_b, rows_b, sem_b)

    @pl.when(wid < EXTRA)
    def _():
        b = (FULL * NW + wid) * C
        pltpu.sync_copy(ei_hbm.at[:, pl.ds(b, C)], idx_a)
        pltpu.async_copy(y_hbm.at[idx_a.at[0]], rows_a, sem_a)
        pltpu.make_async_copy(y_hbm.at[idx_a.at[0]], rows_a, sem_a).wait()
        pltpu.sync_copy(rows_a, acc_sh.at[idx_a.at[1]], add=True)

    plsc.subcore_barrier()
    _copy_out(cid, sid, acc_sh, acc_hbm)


_sc_mesh = plsc.VectorSubcoreMesh(core_axis_name="c", subcore_axis_name="s")

_sc_agg = pl.kernel(
    _sc_agg_body,
    out_type=jax.ShapeDtypeStruct((NC * NP, H), _f32),
    mesh=_sc_mesh,
    scratch_types=[
        pltpu.VMEM((2, C), jnp.int32),
        pltpu.VMEM((C, H), _f32),
        pltpu.VMEM((2, C), jnp.int32),
        pltpu.VMEM((C, H), _f32),
        pltpu.VMEM_SHARED((NP, H), _f32),
        pltpu.SemaphoreType.DMA,
        pltpu.SemaphoreType.DMA,
    ],
)

_DN = (((1,), (1,)), ((), ()))

CB = 12800
GS = E // CB


def _tc_cnt_body(d_ref, o_ref):
    i = pl.program_id(0)
    d = d_ref[0]
    rows = lax.broadcasted_iota(jnp.int32, (128, 1), 0)
    m_hi = (rows == (d >> 7)).astype(jnp.bfloat16)
    m_lo = (rows == (d & 127)).astype(jnp.bfloat16)
    part = lax.dot_general(m_hi, m_lo, (((1,), (1,)), ((), ())),
                           preferred_element_type=_f32)

    @pl.when(i == 0)
    def _():
        o_ref[...] = part

    @pl.when(i > 0)
    def _():
        o_ref[...] += part


_tc_cnt = pl.pallas_call(
    _tc_cnt_body,
    grid=(GS,),
    in_specs=[pl.BlockSpec((1, 1, CB), lambda i: (i, 0, 0))],
    out_specs=pl.BlockSpec((128, 128), lambda i: (0, 0)),
    out_shape=jax.ShapeDtypeStruct((128, 128), _f32),
)


def _tc_pre_body(x_ref, wl_ref, wr_ref, b_ref, y_ref, z_ref):
    x = x_ref[...]
    y_ref[...] = lax.dot_general(x, wl_ref[...], _DN,
                                 preferred_element_type=_f32)
    z_ref[...] = lax.dot_general(x, wr_ref[...], _DN,
                                 preferred_element_type=_f32) + b_ref[...]


def _inv_cnt(cnt_ref):
    return 1.0 / jnp.maximum(cnt_ref[...], 1.0)


def _tc_mid_body(acc_ref, cnt_ref, z_ref, wl_ref, wr_ref, b_ref,
                 y2_ref, z2_ref):
    acc = acc_ref[:N, :] + acc_ref[NP:NP + N, :]
    h = jnp.maximum(acc * _inv_cnt(cnt_ref) + z_ref[...], 0.0)
    y2_ref[...] = lax.dot_general(h, wl_ref[...], _DN,
                                  preferred_element_type=_f32)
    z2_ref[...] = lax.dot_general(h, wr_ref[...], _DN,
                                  preferred_element_type=_f32) + b_ref[...]


def _tc_post_body(acc_ref, cnt_ref, z_ref, o_ref):
    acc = acc_ref[:N, :] + acc_ref[NP:NP + N, :]
    o_ref[...] = acc * _inv_cnt(cnt_ref) + z_ref[...]


_nh = jax.ShapeDtypeStruct((N, H), _f32)

_tc_pre = pl.pallas_call(_tc_pre_body, out_shape=[_nh, _nh])
_tc_mid = pl.pallas_call(_tc_mid_body, out_shape=[_nh, _nh])
_tc_post = pl.pallas_call(_tc_post_body, out_shape=_nh)


@jax.jit
def kernel(x, edge_index, Wl1, Wr1, b1, Wl2, Wr2, b2):
    zrow = jnp.zeros((RCH, H), _f32)

    y1, z1 = _tc_pre(x, Wl1, Wr1, b1.reshape(1, H))
    cntm = _tc_cnt(edge_index[1].reshape(GS, 1, CB))
    cntc = cntm.reshape(128 * 128, 1)[:N]
    p1 = _sc_agg(y1, edge_index, zrow)
    y2, z2 = _tc_mid(p1, cntc, z1, Wl2, Wr2, b2.reshape(1, H))
    p2 = _sc_agg(y2, edge_index, zrow)
    return _tc_post(p2, cntc, z2)


# speedup vs baseline: 1.5471x; 1.0048x over previous
import jax
import jax.numpy as jnp
from jax import lax
from jax.experimental import pallas as pl
from jax.experimental.pallas import tpu as pltpu
from jax.experimental.pallas import tpu_sc as plsc

N = 10000
E = 320000
D = 128
H = 128

NC = 2
NS = 16
NW = NC * NS
C = 128
NCH = E // C
FULL = NCH // NW
EXTRA = NCH - FULL * NW
NP = 10240
RCH = 128
NRCH = NP // RCH
CPT = NRCH // NS

_f32 = jnp.float32


def _copy_out(cid, sid, acc_sh, acc_hbm, sem):
    for k in range(CPT):
        j = sid * CPT + k
        pltpu.async_copy(acc_sh.at[pl.ds(j * RCH, RCH)],
                         acc_hbm.at[pl.ds(cid * NP + j * RCH, RCH)], sem)
    for k in range(CPT):
        j = sid * CPT + k
        pltpu.make_async_copy(acc_sh.at[pl.ds(j * RCH, RCH)],
                              acc_hbm.at[pl.ds(cid * NP + j * RCH, RCH)],
                              sem).wait()


def _sc_agg_body(y_hbm, ei_hbm, zrow_hbm, acc_hbm,
                 idx_a, rows_a, idx_b, rows_b,
                 acc_sh, sem_a, sem_b):
    cid = lax.axis_index("c")
    sid = lax.axis_index("s")
    wid = cid * NS + sid

    def load_and_gather(i, idx_v, rows_v, gsem):
        b = (wid + i * NW) * C
        pltpu.sync_copy(ei_hbm.at[:, pl.ds(b, C)], idx_v)
        pltpu.async_copy(y_hbm.at[idx_v.at[0]], rows_v, gsem)

    def wait_and_scatter(idx_v, rows_v, gsem):
        pltpu.make_async_copy(y_hbm.at[idx_v.at[0]], rows_v, gsem).wait()
        pltpu.sync_copy(rows_v, acc_sh.at[idx_v.at[1]], add=True)

    pltpu.sync_copy(zrow_hbm, rows_a)
    for k in range(CPT):
        j = sid * CPT + k
        pltpu.async_copy(rows_a, acc_sh.at[pl.ds(j * RCH, RCH)], sem_a)
    load_and_gather(0, idx_b, rows_b, sem_b)
    for k in range(CPT):
        j = sid * CPT + k
        pltpu.make_async_copy(rows_a, acc_sh.at[pl.ds(j * RCH, RCH)],
                              sem_a).wait()
    plsc.subcore_barrier()

    def ebody(k, carry):
        i = 2 * k
        load_and_gather(i + 1, idx_a, rows_a, sem_a)
        wait_and_scatter(idx_b, rows_b, sem_b)
        load_and_gather(i + 2, idx_b, rows_b, sem_b)
        wait_and_scatter(idx_a, rows_a, sem_a)
        return carry

    lax.fori_loop(0, (FULL - 2) // 2, ebody, 0)
    load_and_gather(FULL - 1, idx_a, rows_a, sem_a)
    wait_and_scatter(idx_b, rows_b, sem_b)
    wait_and_scatter(idx_a, rows_a, sem_a)

    @pl.when(wid < EXTRA)
    def _():
        b = (FULL * NW + wid) * C
        pltpu.sync_copy(ei_hbm.at[:, pl.ds(b, C)], idx_a)
        pltpu.async_copy(y_hbm.at[idx_a.at[0]], rows_a, sem_a)
        pltpu.make_async_copy(y_hbm.at[idx_a.at[0]], rows_a, sem_a).wait()
        pltpu.sync_copy(rows_a, acc_sh.at[idx_a.at[1]], add=True)

    plsc.subcore_barrier()
    _copy_out(cid, sid, acc_sh, acc_hbm, sem_a)


_sc_mesh = plsc.VectorSubcoreMesh(core_axis_name="c", subcore_axis_name="s")

_sc_agg = pl.kernel(
    _sc_agg_body,
    out_type=jax.ShapeDtypeStruct((NC * NP, H), _f32),
    mesh=_sc_mesh,
    scratch_types=[
        pltpu.VMEM((2, C), jnp.int32),
        pltpu.VMEM((C, H), _f32),
        pltpu.VMEM((2, C), jnp.int32),
        pltpu.VMEM((C, H), _f32),
        pltpu.VMEM_SHARED((NP, H), _f32),
        pltpu.SemaphoreType.DMA,
        pltpu.SemaphoreType.DMA,
    ],
)

_DN = (((1,), (1,)), ((), ()))

CB = 12800
GS = E // CB


def _tc_cnt_body(d_ref, o_ref):
    i = pl.program_id(0)
    d = d_ref[0]
    rows = lax.broadcasted_iota(jnp.int32, (128, 1), 0)
    m_hi = (rows == (d >> 7)).astype(jnp.bfloat16)
    m_lo = (rows == (d & 127)).astype(jnp.bfloat16)
    part = lax.dot_general(m_hi, m_lo, (((1,), (1,)), ((), ())),
                           preferred_element_type=_f32)

    @pl.when(i == 0)
    def _():
        o_ref[...] = part

    @pl.when(i > 0)
    def _():
        o_ref[...] += part


_tc_cnt = pl.pallas_call(
    _tc_cnt_body,
    grid=(GS,),
    in_specs=[pl.BlockSpec((1, 1, CB), lambda i: (i, 0, 0))],
    out_specs=pl.BlockSpec((128, 128), lambda i: (0, 0)),
    out_shape=jax.ShapeDtypeStruct((128, 128), _f32),
)


def _tc_pre_body(x_ref, wl_ref, wr_ref, b_ref, y_ref, z_ref):
    x = x_ref[...]
    y_ref[...] = lax.dot_general(x, wl_ref[...], _DN,
                                 preferred_element_type=_f32)
    z_ref[...] = lax.dot_general(x, wr_ref[...], _DN,
                                 preferred_element_type=_f32) + b_ref[...]


def _inv_cnt(cnt_ref):
    return 1.0 / jnp.maximum(cnt_ref[...], 1.0)


def _tc_mid_body(acc_ref, cnt_ref, z_ref, wl_ref, wr_ref, b_ref,
                 y2_ref, z2_ref):
    acc = acc_ref[:N, :] + acc_ref[NP:NP + N, :]
    h = jnp.maximum(acc * _inv_cnt(cnt_ref) + z_ref[...], 0.0)
    y2_ref[...] = lax.dot_general(h, wl_ref[...], _DN,
                                  preferred_element_type=_f32)
    z2_ref[...] = lax.dot_general(h, wr_ref[...], _DN,
                                  preferred_element_type=_f32) + b_ref[...]


def _tc_post_body(acc_ref, cnt_ref, z_ref, o_ref):
    acc = acc_ref[:N, :] + acc_ref[NP:NP + N, :]
    o_ref[...] = acc * _inv_cnt(cnt_ref) + z_ref[...]


_nh = jax.ShapeDtypeStruct((N, H), _f32)

_tc_pre = pl.pallas_call(_tc_pre_body, out_shape=[_nh, _nh])
_tc_mid = pl.pallas_call(_tc_mid_body, out_shape=[_nh, _nh])
_tc_post = pl.pallas_call(_tc_post_body, out_shape=_nh)


@jax.jit
def kernel(x, edge_index, Wl1, Wr1, b1, Wl2, Wr2, b2):
    zrow = jnp.zeros((RCH, H), _f32)

    y1, z1 = _tc_pre(x, Wl1, Wr1, b1.reshape(1, H))
    cntm = _tc_cnt(edge_index[1].reshape(GS, 1, CB))
    cntc = cntm.reshape(128 * 128, 1)[:N]
    p1 = _sc_agg(y1, edge_index, zrow)
    y2, z2 = _tc_mid(p1, cntc, z1, Wl2, Wr2, b2.reshape(1, H))
    p2 = _sc_agg(y2, edge_index, zrow)
    return _tc_post(p2, cntc, z2)
